# baseline (speedup 1.0000x reference)
import functools

import jax
import jax.numpy as jnp
from jax import lax
from jax.experimental import pallas as pl
from jax.experimental.pallas import tpu as pltpu
from jax.experimental.pallas import tpu_sc as plsc

N_FIELDS = 26
N_DENSE = 13
VOCAB = 100000
DIM = 32
BATCH = 16384

NC, NS = 2, 16
NW = NC * NS
ROWS_PER_W = BATCH // NW
R = 128
N_CHUNKS = ROWS_PER_W // R

_mesh = plsc.VectorSubcoreMesh(
    core_axis_name="c", subcore_axis_name="s", num_cores=NC, num_subcores=NS
)


@functools.partial(
    pl.kernel,
    out_type=jax.ShapeDtypeStruct((BATCH * N_FIELDS, DIM), jnp.float32),
    mesh=_mesh,
    scratch_types=[
        pltpu.VMEM((N_FIELDS, R), jnp.int32),
        pltpu.VMEM((N_FIELDS * R, DIM), jnp.float32),
        pltpu.SemaphoreType.DMA,
    ],
    compiler_params=pltpu.CompilerParams(use_tc_tiling_on_sc=False),
)
def _gather_emb(idx_hbm, tbl_hbm, emb_hbm, idx_v, cont_v, sem):
    wid = lax.axis_index("s") * NC + lax.axis_index("c")

    @pl.loop(0, N_CHUNKS)
    def _chunk(c):
        k = wid * N_CHUNKS + c
        pltpu.sync_copy(idx_hbm.at[pl.ds(k * N_FIELDS, N_FIELDS), :], idx_v)
        descs = [
            pltpu.async_copy(
                tbl_hbm.at[idx_v.at[g]], cont_v.at[pl.ds(g * R, R), :], sem
            )
            for g in range(N_FIELDS)
        ]
        for d in descs:
            d.wait()
        pltpu.sync_copy(
            cont_v, emb_hbm.at[pl.ds(k * N_FIELDS * R, N_FIELDS * R), :]
        )


def kernel(indices, dense, tables):
    offs = (jnp.arange(N_FIELDS, dtype=jnp.int32) * VOCAB)[:, None]
    idx_int = (indices + offs).T.reshape(BATCH * N_FIELDS // R, R)
    flat_tbl = tables.reshape(N_FIELDS * VOCAB, DIM)
    emb = _gather_emb(idx_int, flat_tbl)
    return jnp.concatenate(
        [emb.reshape(BATCH, N_FIELDS * DIM), dense.T], axis=1
    )


# speedup vs baseline: 1.0060x; 1.0060x over previous
import functools

import jax
import jax.numpy as jnp
from jax import lax
from jax.experimental import pallas as pl
from jax.experimental.pallas import tpu as pltpu
from jax.experimental.pallas import tpu_sc as plsc

N_FIELDS = 26
N_DENSE = 13
VOCAB = 100000
DIM = 32
BATCH = 16384

NC, NS = 2, 16
NW = NC * NS
ROWS_PER_W = BATCH // NW
R = 128
N_CHUNKS = ROWS_PER_W // R

_mesh = plsc.VectorSubcoreMesh(
    core_axis_name="c", subcore_axis_name="s", num_cores=NC, num_subcores=NS
)


@functools.partial(
    pl.kernel,
    out_type=jax.ShapeDtypeStruct((BATCH * N_FIELDS, DIM), jnp.float32),
    mesh=_mesh,
    scratch_types=[
        pltpu.VMEM((N_FIELDS, R), jnp.int32),
        pltpu.VMEM((N_FIELDS * R, DIM), jnp.float32),
        pltpu.SemaphoreType.DMA,
    ],
    compiler_params=pltpu.CompilerParams(use_tc_tiling_on_sc=False),
)
def _gather_emb(idx_hbm, tbl_hbm, emb_hbm, idx_v, cont_v, sem):
    wid = lax.axis_index("s") * NC + lax.axis_index("c")

    @pl.loop(0, N_CHUNKS)
    def _chunk(c):
        k = wid * N_CHUNKS + c
        pltpu.sync_copy(idx_hbm.at[pl.ds(k * N_FIELDS, N_FIELDS), :], idx_v)
        descs = [
            pltpu.async_copy(
                tbl_hbm.at[idx_v.at[g]], cont_v.at[pl.ds(g * R, R), :], sem
            )
            for g in range(N_FIELDS)
        ]
        for d in descs:
            d.wait()
        pltpu.sync_copy(
            cont_v, emb_hbm.at[pl.ds(k * N_FIELDS * R, N_FIELDS * R), :]
        )


def kernel(indices, dense, tables):
    offs = (jnp.arange(N_FIELDS, dtype=jnp.int32) * VOCAB)[:, None]
    idx_int = (indices + offs).T.reshape(BATCH * N_FIELDS // R, R)
    flat_tbl = tables.reshape(N_FIELDS * VOCAB, DIM)
    emb = _gather_emb(idx_int, flat_tbl)
    one = (indices[0, 0] >> jnp.int32(31)).astype(jnp.float32) + 1.0
    return (
        jnp.concatenate([emb.reshape(BATCH, N_FIELDS * DIM), dense.T], axis=1)
        * one
    )
